# baseline (speedup 1.0000x reference)
import functools

import jax
import jax.numpy as jnp
from jax import lax
from jax.experimental import pallas as pl
from jax.experimental.pallas import tpu as pltpu
from jax.experimental.pallas import tpu_sc as plsc

N = 10000
E = 320000
NFEAT = 128
NHID = 128

NPAD = 10240
NTILE = 16
NODES_PER_TILE = NPAD // NTILE
EROWS_P = 2560
EPAD = EROWS_P * 128
TILE_ROWS = 160
NVEC = NODES_PER_TILE // 16
LAG = 8

_TC_BLOCK = 2048
_TC_GRID = NPAD // _TC_BLOCK


def _tc_body(x_ref, xs_ref, we1_ref, we2_ref, wg_ref, wc_ref,
             be1_ref, be2_ref, bg_ref, bc_ref,
             v_ref, u_ref, cs_ref, cy_ref):
    we = we1_ref[...] @ we2_ref[...]
    wg = wg_ref[...] @ wc_ref[...]
    v_ref[...] = x_ref[...] @ we
    u_ref[...] = xs_ref[...] @ wg

    @pl.when(pl.program_id(0) == 0)
    def _():
        cs_ref[...] = be1_ref[...] @ we2_ref[...] + be2_ref[...]
        cy_ref[...] = bg_ref[...] @ wc_ref[...] + bc_ref[...]


def _tc_matvecs(x, xs, W_e1, W_e2, W_g, W_c, b_e1, b_e2, b_g, b_c):
    full = lambda shape: pl.BlockSpec(shape, lambda i: (0, 0))
    return pl.pallas_call(
        _tc_body,
        grid=(_TC_GRID,),
        in_specs=[
            pl.BlockSpec((_TC_BLOCK, NFEAT), lambda i: (i, 0)),
            pl.BlockSpec((_TC_BLOCK, NFEAT + 1), lambda i: (i, 0)),
            full((NFEAT, NHID)),
            full((NHID, 1)),
            full((NFEAT + 1, NHID)),
            full((NHID, 1)),
            full((1, NHID)),
            full((1, 1)),
            full((1, NHID)),
            full((1, 1)),
        ],
        out_specs=[
            pl.BlockSpec((_TC_BLOCK, 1), lambda i: (i, 0)),
            pl.BlockSpec((_TC_BLOCK, 1), lambda i: (i, 0)),
            full((1, 1)),
            full((1, 1)),
        ],
        out_shape=[
            jax.ShapeDtypeStruct((NPAD, 1), jnp.float32),
            jax.ShapeDtypeStruct((NPAD, 1), jnp.float32),
            jax.ShapeDtypeStruct((1, 1), jnp.float32),
            jax.ShapeDtypeStruct((1, 1), jnp.float32),
        ],
    )(x, xs, W_e1, W_e2, W_g, W_c,
      b_e1.reshape(1, NHID), b_e2.reshape(1, 1),
      b_g.reshape(1, NHID), b_c.reshape(1, 1))


def _rsqrt_or_zero(d):
    x = jnp.maximum(1.0, d * 0.0625)
    for _ in range(10):
        x = 0.5 * (x + d / x)
    return jnp.where(d > 0.5, 1.0 / x, 0.0)


def _sc_body(e_hbm, v_hbm, u_hbm,
             s_hbm, y_hbm,
             sh_odeg, sh_ideg, sh_val, sh_acc,
             src_full, dst_full, val_local, msg,
             ones_row, nbuf_a, nbuf_b, ndst_buf, sem):
    cid = lax.axis_index("c")
    wid = lax.axis_index("s")
    sl = pl.ds(wid * NODES_PER_TILE, NODES_PER_TILE)
    row0 = wid * TILE_ROWS
    lo = 0
    hi = TILE_ROWS

    def _zero16(i, _):
        nbuf_a[pl.ds(i * 16, 16)] = jnp.zeros((16,), jnp.float32)
        return 0
    lax.fori_loop(0, NVEC, _zero16, 0)
    for c in range(8):
        ones_row[pl.ds(c * 16, 16)] = jnp.full((16,), 1.0, jnp.float32)

    pltpu.sync_copy(e_hbm.at[0, pl.ds(row0, TILE_ROWS)], src_full)
    pltpu.sync_copy(e_hbm.at[1, pl.ds(row0, TILE_ROWS)], dst_full)

    pltpu.sync_copy(nbuf_a, sh_odeg.at[sl])
    pltpu.sync_copy(nbuf_a, sh_ideg.at[sl])
    pltpu.sync_copy(nbuf_a, sh_acc.at[sl])
    plsc.subcore_barrier()

    def _deg_row(r, _):
        pltpu.async_copy(ones_row, sh_odeg.at[src_full.at[r]], sem, add=True)
        pltpu.async_copy(ones_row, sh_ideg.at[dst_full.at[r]], sem, add=True)

        @pl.when(r >= lo + LAG)
        def _():
            pltpu.make_async_copy(
                ones_row, sh_odeg.at[src_full.at[r - LAG]], sem).wait()
            pltpu.make_async_copy(
                ones_row, sh_ideg.at[dst_full.at[r - LAG]], sem).wait()
        return 0
    lax.fori_loop(lo, hi, _deg_row, 0)

    def _deg_drain(i, _):
        r = hi - LAG + i
        pltpu.make_async_copy(ones_row, sh_odeg.at[src_full.at[r]], sem).wait()
        pltpu.make_async_copy(ones_row, sh_ideg.at[dst_full.at[r]], sem).wait()
        return 0
    lax.fori_loop(0, LAG, _deg_drain, 0)
    plsc.subcore_barrier()

    pltpu.sync_copy(sh_odeg.at[sl], nbuf_a)

    def _nsrc16(i, _):
        ix = pl.ds(i * 16, 16)
        nbuf_a[ix] = _rsqrt_or_zero(nbuf_a[ix])
        return 0
    lax.fori_loop(0, NVEC, _nsrc16, 0)

    @pl.when(cid == 0)
    def _():
        pltpu.sync_copy(v_hbm.at[sl], nbuf_b)

    @pl.when(cid == 1)
    def _():
        pltpu.sync_copy(u_hbm.at[sl], nbuf_b)

    def _pv16(i, _):
        ix = pl.ds(i * 16, 16)
        nbuf_b[ix] = nbuf_b[ix] * nbuf_a[ix]
        return 0
    lax.fori_loop(0, NVEC, _pv16, 0)
    pltpu.sync_copy(nbuf_b, sh_val.at[sl])

    pltpu.sync_copy(sh_ideg.at[sl], ndst_buf)

    def _ndst16(i, _):
        ix = pl.ds(i * 16, 16)
        ndst_buf[ix] = _rsqrt_or_zero(ndst_buf[ix])
        return 0
    lax.fori_loop(0, NVEC, _ndst16, 0)
    plsc.subcore_barrier()

    pltpu.sync_copy(sh_val, val_local)

    def _edge_row(r, _):
        for c in range(8):
            ix = pl.ds(c * 16, 16)
            idx = src_full[r, ix]
            msg[r, ix] = plsc.load_gather(val_local, [idx])
        pltpu.async_copy(msg.at[r], sh_acc.at[dst_full.at[r]], sem, add=True)

        @pl.when(r >= lo + LAG)
        def _():
            pltpu.make_async_copy(
                msg.at[r - LAG], sh_acc.at[dst_full.at[r - LAG]], sem).wait()
        return 0
    lax.fori_loop(lo, hi, _edge_row, 0)

    def _edge_drain(i, _):
        r = hi - LAG + i
        pltpu.make_async_copy(msg.at[r], sh_acc.at[dst_full.at[r]], sem).wait()
        return 0
    lax.fori_loop(0, LAG, _edge_drain, 0)
    plsc.subcore_barrier()

    pltpu.sync_copy(sh_acc.at[sl], nbuf_a)

    def _scale16(i, _):
        ix = pl.ds(i * 16, 16)
        nbuf_a[ix] = nbuf_a[ix] * ndst_buf[ix]
        return 0
    lax.fori_loop(0, NVEC, _scale16, 0)

    @pl.when(cid == 0)
    def _():
        pltpu.sync_copy(nbuf_a, s_hbm.at[sl])

    @pl.when(cid == 1)
    def _():
        pltpu.sync_copy(nbuf_a, y_hbm.at[sl])


_sc_graph = functools.partial(
    pl.kernel,
    out_type=[
        jax.ShapeDtypeStruct((NPAD,), jnp.float32),
        jax.ShapeDtypeStruct((NPAD,), jnp.float32),
    ],
    mesh=plsc.VectorSubcoreMesh(
        core_axis_name="c", subcore_axis_name="s", num_cores=2),
    compiler_params=pltpu.CompilerParams(needs_layout_passes=False),
    scratch_types=[
        pltpu.VMEM_SHARED((NPAD,), jnp.float32),
        pltpu.VMEM_SHARED((NPAD,), jnp.float32),
        pltpu.VMEM_SHARED((NPAD,), jnp.float32),
        pltpu.VMEM_SHARED((NPAD,), jnp.float32),
        pltpu.VMEM((TILE_ROWS, 128), jnp.int32),
        pltpu.VMEM((TILE_ROWS, 128), jnp.int32),
        pltpu.VMEM((NPAD,), jnp.float32),
        pltpu.VMEM((TILE_ROWS, 128), jnp.float32),
        pltpu.VMEM((128,), jnp.float32),
        pltpu.VMEM((NODES_PER_TILE,), jnp.float32),
        pltpu.VMEM((NODES_PER_TILE,), jnp.float32),
        pltpu.VMEM((NODES_PER_TILE,), jnp.float32),
        pltpu.SemaphoreType.DMA,
    ],
)(_sc_body)


@jax.jit
def kernel(x, x_sens, edge_index, W_e1, b_e1, W_e2, b_e2, W_g, b_g, W_c, b_c):
    v2d, u2d, cs, cy = _tc_matvecs(x, x_sens, W_e1, W_e2, W_g, W_c,
                                   b_e1, b_e2, b_g, b_c)

    e3d = jnp.pad(edge_index, ((0, 0), (0, EPAD - E)),
                  constant_values=NPAD - 1).reshape(2, EROWS_P, 128)

    s_raw, y_raw = _sc_graph(e3d, v2d.reshape(NPAD), u2d.reshape(NPAD))

    s = (s_raw[:N] + cs[0, 0]).reshape(N, 1)
    y = (y_raw[:N] + cy[0, 0]).reshape(N, 1)
    return (y, s)


# speedup vs baseline: 1.1965x; 1.1965x over previous
import functools

import jax
import jax.numpy as jnp
from jax import lax
from jax.experimental import pallas as pl
from jax.experimental.pallas import tpu as pltpu
from jax.experimental.pallas import tpu_sc as plsc

N = 10000
E = 320000
NFEAT = 128
NHID = 128

NPAD = 10240
NTILE = 16
NODES_PER_TILE = NPAD // NTILE
EROWS = E // 128
ROWS_BASE = EROWS // NTILE
ROWS_EXTRA = EROWS - ROWS_BASE * NTILE
ROWS_MAX = ROWS_BASE + 1
NVEC = NODES_PER_TILE // 16
LAG = 8

_TC_BLOCK = 2048
_TC_GRID = NPAD // _TC_BLOCK


def _tc_body(x_ref, xs_ref, we1_ref, we2_ref, wg_ref, wc_ref,
             be1_ref, be2_ref, bg_ref, bc_ref,
             v_ref, u_ref, cs_ref, cy_ref):
    we = we1_ref[...] @ we2_ref[...]
    wg = wg_ref[...] @ wc_ref[...]
    v_ref[...] = x_ref[...] @ we
    u_ref[...] = xs_ref[...] @ wg

    @pl.when(pl.program_id(0) == 0)
    def _():
        cs_ref[...] = be1_ref[...] @ we2_ref[...] + be2_ref[...]
        cy_ref[...] = bg_ref[...] @ wc_ref[...] + bc_ref[...]


def _tc_matvecs(x, xs, W_e1, W_e2, W_g, W_c, b_e1, b_e2, b_g, b_c):
    full = lambda shape: pl.BlockSpec(shape, lambda i: (0, 0))
    return pl.pallas_call(
        _tc_body,
        grid=(_TC_GRID,),
        in_specs=[
            pl.BlockSpec((_TC_BLOCK, NFEAT), lambda i: (i, 0)),
            pl.BlockSpec((_TC_BLOCK, NFEAT + 1), lambda i: (i, 0)),
            full((NFEAT, NHID)),
            full((NHID, 1)),
            full((NFEAT + 1, NHID)),
            full((NHID, 1)),
            full((1, NHID)),
            full((1, 1)),
            full((1, NHID)),
            full((1, 1)),
        ],
        out_specs=[
            pl.BlockSpec((_TC_BLOCK, 1), lambda i: (i, 0)),
            pl.BlockSpec((_TC_BLOCK, 1), lambda i: (i, 0)),
            full((1, 1)),
            full((1, 1)),
        ],
        out_shape=[
            jax.ShapeDtypeStruct((NPAD, 1), jnp.float32),
            jax.ShapeDtypeStruct((NPAD, 1), jnp.float32),
            jax.ShapeDtypeStruct((1, 1), jnp.float32),
            jax.ShapeDtypeStruct((1, 1), jnp.float32),
        ],
    )(x, xs, W_e1, W_e2, W_g, W_c,
      b_e1.reshape(1, NHID), b_e2.reshape(1, 1),
      b_g.reshape(1, NHID), b_c.reshape(1, 1))


def _rsqrt_or_zero(d):
    x = jnp.maximum(1.0, d * 0.0625)
    for _ in range(10):
        x = 0.5 * (x + d / x)
    return jnp.where(d > 0.5, 1.0 / x, 0.0)


def _sc_body(e_hbm, v_hbm, u_hbm,
             s_hbm, y_hbm,
             sh_odeg, sh_ideg, sh_val, sh_acc,
             src_full, dst_full, val_local, msg,
             ones_row, nbuf_a, nbuf_b, ndst_buf, sem):
    cid = lax.axis_index("c")
    wid = lax.axis_index("s")
    sl = pl.ds(wid * NODES_PER_TILE, NODES_PER_TILE)
    row0 = wid * ROWS_BASE + jnp.minimum(wid, ROWS_EXTRA)
    nrows = ROWS_BASE + jnp.where(wid < ROWS_EXTRA, 1, 0)
    row0c = jnp.minimum(row0, EROWS - ROWS_MAX)
    off = row0 - row0c
    lo = off
    hi = off + nrows

    def _zero16(i, _):
        nbuf_a[pl.ds(i * 16, 16)] = jnp.zeros((16,), jnp.float32)
        return 0
    lax.fori_loop(0, NVEC, _zero16, 0)
    for c in range(8):
        ones_row[pl.ds(c * 16, 16)] = jnp.full((16,), 1.0, jnp.float32)

    pltpu.sync_copy(e_hbm.at[0, pl.ds(row0c, ROWS_MAX)], src_full)
    pltpu.sync_copy(e_hbm.at[1, pl.ds(row0c, ROWS_MAX)], dst_full)

    pltpu.sync_copy(nbuf_a, sh_odeg.at[sl])
    pltpu.sync_copy(nbuf_a, sh_ideg.at[sl])
    pltpu.sync_copy(nbuf_a, sh_acc.at[sl])
    plsc.subcore_barrier()

    def _deg_row(r, _):
        pltpu.async_copy(ones_row, sh_odeg.at[src_full.at[r]], sem, add=True)
        pltpu.async_copy(ones_row, sh_ideg.at[dst_full.at[r]], sem, add=True)

        @pl.when(r >= lo + LAG)
        def _():
            pltpu.make_async_copy(
                ones_row, sh_odeg.at[src_full.at[r - LAG]], sem).wait()
            pltpu.make_async_copy(
                ones_row, sh_ideg.at[dst_full.at[r - LAG]], sem).wait()
        return 0
    lax.fori_loop(lo, hi, _deg_row, 0)

    def _deg_drain(i, _):
        r = hi - LAG + i
        pltpu.make_async_copy(ones_row, sh_odeg.at[src_full.at[r]], sem).wait()
        pltpu.make_async_copy(ones_row, sh_ideg.at[dst_full.at[r]], sem).wait()
        return 0
    lax.fori_loop(0, LAG, _deg_drain, 0)
    plsc.subcore_barrier()

    pltpu.sync_copy(sh_odeg.at[sl], nbuf_a)

    def _nsrc16(i, _):
        ix = pl.ds(i * 16, 16)
        nbuf_a[ix] = _rsqrt_or_zero(nbuf_a[ix])
        return 0
    lax.fori_loop(0, NVEC, _nsrc16, 0)

    @pl.when(cid == 0)
    def _():
        pltpu.sync_copy(v_hbm.at[sl], nbuf_b)

    @pl.when(cid == 1)
    def _():
        pltpu.sync_copy(u_hbm.at[sl], nbuf_b)

    def _pv16(i, _):
        ix = pl.ds(i * 16, 16)
        nbuf_b[ix] = nbuf_b[ix] * nbuf_a[ix]
        return 0
    lax.fori_loop(0, NVEC, _pv16, 0)
    pltpu.sync_copy(nbuf_b, sh_val.at[sl])

    pltpu.sync_copy(sh_ideg.at[sl], ndst_buf)

    def _ndst16(i, _):
        ix = pl.ds(i * 16, 16)
        ndst_buf[ix] = _rsqrt_or_zero(ndst_buf[ix])
        return 0
    lax.fori_loop(0, NVEC, _ndst16, 0)
    plsc.subcore_barrier()

    pltpu.sync_copy(sh_val, val_local)

    def _edge_row(r, _):
        for c in range(8):
            ix = pl.ds(c * 16, 16)
            idx = src_full[r, ix]
            msg[r, ix] = plsc.load_gather(val_local, [idx])
        pltpu.async_copy(msg.at[r], sh_acc.at[dst_full.at[r]], sem, add=True)

        @pl.when(r >= lo + LAG)
        def _():
            pltpu.make_async_copy(
                msg.at[r - LAG], sh_acc.at[dst_full.at[r - LAG]], sem).wait()
        return 0
    lax.fori_loop(lo, hi, _edge_row, 0)

    def _edge_drain(i, _):
        r = hi - LAG + i
        pltpu.make_async_copy(msg.at[r], sh_acc.at[dst_full.at[r]], sem).wait()
        return 0
    lax.fori_loop(0, LAG, _edge_drain, 0)
    plsc.subcore_barrier()

    pltpu.sync_copy(sh_acc.at[sl], nbuf_a)

    def _scale16(i, _):
        ix = pl.ds(i * 16, 16)
        nbuf_a[ix] = nbuf_a[ix] * ndst_buf[ix]
        return 0
    lax.fori_loop(0, NVEC, _scale16, 0)

    @pl.when(cid == 0)
    def _():
        pltpu.sync_copy(nbuf_a, s_hbm.at[sl])

    @pl.when(cid == 1)
    def _():
        pltpu.sync_copy(nbuf_a, y_hbm.at[sl])


_sc_graph = functools.partial(
    pl.kernel,
    out_type=[
        jax.ShapeDtypeStruct((NPAD,), jnp.float32),
        jax.ShapeDtypeStruct((NPAD,), jnp.float32),
    ],
    mesh=plsc.VectorSubcoreMesh(
        core_axis_name="c", subcore_axis_name="s", num_cores=2),
    compiler_params=pltpu.CompilerParams(
        needs_layout_passes=False, use_tc_tiling_on_sc=False),
    scratch_types=[
        pltpu.VMEM_SHARED((NPAD,), jnp.float32),
        pltpu.VMEM_SHARED((NPAD,), jnp.float32),
        pltpu.VMEM_SHARED((NPAD,), jnp.float32),
        pltpu.VMEM_SHARED((NPAD,), jnp.float32),
        pltpu.VMEM((ROWS_MAX, 128), jnp.int32),
        pltpu.VMEM((ROWS_MAX, 128), jnp.int32),
        pltpu.VMEM((NPAD,), jnp.float32),
        pltpu.VMEM((ROWS_MAX, 128), jnp.float32),
        pltpu.VMEM((128,), jnp.float32),
        pltpu.VMEM((NODES_PER_TILE,), jnp.float32),
        pltpu.VMEM((NODES_PER_TILE,), jnp.float32),
        pltpu.VMEM((NODES_PER_TILE,), jnp.float32),
        pltpu.SemaphoreType.DMA,
    ],
)(_sc_body)


@jax.jit
def kernel(x, x_sens, edge_index, W_e1, b_e1, W_e2, b_e2, W_g, b_g, W_c, b_c):
    v2d, u2d, cs, cy = _tc_matvecs(x, x_sens, W_e1, W_e2, W_g, W_c,
                                   b_e1, b_e2, b_g, b_c)

    e3d = edge_index.reshape(2, EROWS, 128)

    s_raw, y_raw = _sc_graph(e3d, v2d.reshape(NPAD), u2d.reshape(NPAD))

    s = (s_raw[:N] + cs[0, 0]).reshape(N, 1)
    y = (y_raw[:N] + cy[0, 0]).reshape(N, 1)
    return (y, s)
